# baseline (device time: 19355 ns/iter reference)
import jax
import jax.numpy as jnp
from jax import lax
from jax.experimental import pallas as pl
from jax.experimental.pallas import tpu as pltpu

N_DEV = 4
CHUNK = 256
N_CHUNKS = 8
N_BUF = 6


def kernel(x, dy, gamma):
    del gamma
    m_per, d = x.shape
    n_chunks = m_per // CHUNK
    assert n_chunks == N_CHUNKS

    def body(x_hbm, dy_hbm, out_ref, xbuf, dybuf, acc_ref, comm_ref,
             xsems, dysems, send_sems, recv_sems):
        my_pos = lax.axis_index("i")
        barrier_sem = pltpu.get_barrier_semaphore()
        for k in range(1, N_DEV):
            pl.semaphore_signal(
                barrier_sem, inc=1,
                device_id=((my_pos + k) % N_DEV,),
                device_id_type=pl.DeviceIdType.MESH,
            )

        def dma_pair(c):
            s = c % N_BUF
            xd = pltpu.make_async_copy(
                x_hbm.at[pl.ds(c * CHUNK, CHUNK), :], xbuf.at[s], xsems.at[s])
            dd = pltpu.make_async_copy(
                dy_hbm.at[pl.ds(c * CHUNK, CHUNK), :], dybuf.at[s], dysems.at[s])
            return xd, dd

        for c in range(N_BUF):
            xd, dd = dma_pair(c)
            xd.start()
            dd.start()

        for c in range(N_CHUNKS):
            s = c % N_BUF
            xd, dd = dma_pair(c)
            xd.wait()
            dd.wait()
            xv = xbuf[s]
            dyv = dybuf[s]
            mu = jnp.mean(xv, axis=1, keepdims=True)
            var = jnp.mean(xv * xv, axis=1, keepdims=True) - mu * mu
            rstd = lax.rsqrt(var + 1e-5)
            xhat = (xv - mu) * rstd
            q = dyv * xhat
            ones_row = jnp.ones((1, CHUNK), jnp.float32)
            dgamma = jnp.dot(ones_row, q, preferred_element_type=jnp.float32)
            dbeta = jnp.dot(ones_row, dyv, preferred_element_type=jnp.float32)
            partial = jnp.concatenate([dgamma, dbeta], axis=0)
            if c == 0:
                acc_ref[:, :] = partial
            else:
                acc_ref[:, :] = acc_ref[:, :] + partial
            if c + N_BUF < N_CHUNKS:
                xd2, dd2 = dma_pair(c + N_BUF)
                xd2.start()
                dd2.start()

        pl.semaphore_wait(barrier_sem, N_DEV - 1)
        comm_ref[N_DEV - 1, :, :] = acc_ref[:, :]
        rdmas = []
        for k in range(1, N_DEV):
            rdma = pltpu.make_async_remote_copy(
                src_ref=comm_ref.at[N_DEV - 1],
                dst_ref=comm_ref.at[k - 1],
                send_sem=send_sems.at[k - 1],
                recv_sem=recv_sems.at[k - 1],
                device_id=((my_pos + k) % N_DEV,),
                device_id_type=pl.DeviceIdType.MESH,
            )
            rdma.start()
            rdmas.append(rdma)
        for rdma in rdmas:
            rdma.wait_recv()
        out_ref[:, :] = (
            comm_ref[0, :, :] + comm_ref[1, :, :]
            + comm_ref[2, :, :] + comm_ref[3, :, :]
        )
        for rdma in rdmas:
            rdma.wait_send()

    return pl.pallas_call(
        body,
        out_shape=jax.ShapeDtypeStruct((2, d), jnp.float32),
        in_specs=[
            pl.BlockSpec(memory_space=pl.ANY),
            pl.BlockSpec(memory_space=pl.ANY),
        ],
        out_specs=pl.BlockSpec(memory_space=pltpu.VMEM),
        scratch_shapes=[
            pltpu.VMEM((N_BUF, CHUNK, d), jnp.float32),
            pltpu.VMEM((N_BUF, CHUNK, d), jnp.float32),
            pltpu.VMEM((2, d), jnp.float32),
            pltpu.VMEM((N_DEV, 2, d), jnp.float32),
            pltpu.SemaphoreType.DMA((N_BUF,)),
            pltpu.SemaphoreType.DMA((N_BUF,)),
            pltpu.SemaphoreType.DMA((N_DEV - 1,)),
            pltpu.SemaphoreType.DMA((N_DEV - 1,)),
        ],
        compiler_params=pltpu.CompilerParams(collective_id=0),
    )(x, dy)
